# baseline (speedup 1.0000x reference)
import functools

import jax
import jax.numpy as jnp
from jax import lax
from jax.experimental import pallas as pl
from jax.experimental.pallas import tpu as pltpu
from jax.experimental.pallas import tpu_sc as plsc

B, K, H, W = 8, 100, 128, 128
HW = H * W
C = 7
SPT = 25
UNROLL = 16
NITER = HW // (16 * UNROLL)


def _merge(mv, iv, mv2, iv2):
    take = (mv2 > mv) | ((mv2 == mv) & (iv2 < iv))
    return jnp.where(take, mv2, mv), jnp.where(take, iv2, iv)


NBUF = 4


def _sc_body(hm_hbm, pred_hbm, gt_hbm, out_hbm, parts_hbm,
             hbuf, vms_v, idxs_v, gt_v, val_v,
             part_v, fold_v, red_v, out_v, sem0, sem1, sem2, sem3, semg):
    cid = lax.axis_index("c")
    sid = lax.axis_index("s")
    w = sid * 2 + cid
    base = w * SPT
    b_img = w // 4
    it = lax.broadcasted_iota(jnp.int32, (16,), 0)
    sems = (sem0, sem1, sem2, sem3)

    def row_copy(r):
        return pltpu.async_copy(
            hm_hbm.at[pl.ds((base + r) * HW, HW)], hbuf.at[r % NBUF],
            sems[r % NBUF])

    pending = {r: row_copy(r) for r in range(NBUF)}
    neg_inf = jnp.full((16,), -jnp.inf, jnp.float32)
    zero_i = jnp.zeros((16,), jnp.int32)
    vm_g = [jnp.zeros((16,), jnp.float32)] * 2
    pb_g = [zero_i] * 2

    idx0 = base + it
    idx1 = base + jnp.minimum(16 + it, SPT - 1)
    copies = []
    for g, idxg in enumerate((idx0, idx1)):
        for c in range(C):
            copies.append(
                pltpu.async_copy(gt_hbm.at[idxg * 8 + c], gt_v.at[g * C + c], semg))

    for r in range(SPT):
        pending.pop(r).wait()
        if r + NBUF < SPT:
            pending[r + NBUF] = row_copy(r + NBUF)

        ms = [neg_inf] * UNROLL
        ids = [zero_i] * UNROLL

        def body(i, carry):
            ms, ids = carry
            ms, ids = list(ms), list(ids)
            iv = jnp.broadcast_to(i, (16,)).astype(jnp.int32)
            off = i * (16 * UNROLL)
            for j in range(UNROLL):
                v = hbuf[r % NBUF, pl.ds(off + j * 16, 16)]
                take = v > ms[j]
                ms[j] = jnp.where(take, v, ms[j])
                ids[j] = jnp.where(take, iv, ids[j])
            return tuple(ms), tuple(ids)

        ms, ids = lax.fori_loop(0, NITER, body, (tuple(ms), tuple(ids)))
        mv, iv = ms[0], ids[0] * (16 * UNROLL) + it
        for j in range(1, UNROLL):
            mv, iv = _merge(mv, iv, ms[j],
                            ids[j] * (16 * UNROLL) + (j * 16) + it)

        vms_v[pl.ds(0, 16)] = mv
        idxs_v[pl.ds(0, 16)] = iv
        for sh in (8, 4, 2, 1):
            av = vms_v[pl.ds(0, 16)]
            ai = idxs_v[pl.ds(0, 16)]
            bv = vms_v[pl.ds(sh, 16)]
            bi = idxs_v[pl.ds(sh, 16)]
            av, ai = _merge(av, ai, bv, bi)
            vms_v[pl.ds(0, 16)] = av
            idxs_v[pl.ds(0, 16)] = ai
        mval = vms_v[pl.ds(0, 16)][0]
        midx = idxs_v[pl.ds(0, 16)][0] + b_img * (8 * HW)
        g, lane = (0, r) if r < 16 else (1, r - 16)
        vm_g[g] = jnp.where(it == lane, jnp.broadcast_to(mval, (16,)), vm_g[g])
        pb_g[g] = jnp.where(it == lane, jnp.broadcast_to(midx, (16,)), pb_g[g])
        if r in (15, SPT - 1):
            for c in range(C):
                copies.append(
                    pltpu.async_copy(pred_hbm.at[pb_g[g] + c * HW],
                                     val_v.at[g * C + c], semg))

    for cp in copies:
        cp.wait()

    total = jnp.zeros((16,), jnp.float32)
    for g in range(2):
        acc = jnp.zeros((16,), jnp.float32)
        for c in range(C):
            d = val_v[g * C + c] - gt_v[g * C + c]
            acc = acc + d * d
        live = vm_g[g] == 1.0
        if g == 1:
            live = live & (it < SPT - 16)
        total = total + jnp.where(live, acc, jnp.float32(0.0))

    part_v[pl.ds(0, 16)] = total
    for sh in (8, 4, 2, 1):
        part_v[pl.ds(0, 16)] = part_v[pl.ds(0, 16)] + part_v[pl.ds(sh, 16)]
    fold_v[...] = part_v[pl.ds(0, 16)]
    pltpu.sync_copy(fold_v, parts_hbm.at[w])
    plsc.subcore_barrier()

    @pl.when(w == 0)
    def _():
        pltpu.sync_copy(parts_hbm, red_v)
        vec = jnp.zeros((16,), jnp.float32)
        for bb in range(B):
            rs = (red_v[4 * bb] + red_v[4 * bb + 1]
                  + red_v[4 * bb + 2] + red_v[4 * bb + 3])
            vec = jnp.where(it == bb, jnp.broadcast_to(rs[0], (16,)), vec)
        out_v[...] = vec
        pltpu.sync_copy(out_v, out_hbm)


@functools.cache
def _sc_kernel():
    mesh = plsc.VectorSubcoreMesh(core_axis_name="c", subcore_axis_name="s")
    return pl.kernel(
        _sc_body,
        out_type=(jax.ShapeDtypeStruct((16,), jnp.float32),
                  jax.ShapeDtypeStruct((32, 16), jnp.float32)),
        mesh=mesh,
        scratch_types=[
            pltpu.VMEM((NBUF, HW), jnp.float32),
            pltpu.VMEM((32,), jnp.float32),
            pltpu.VMEM((32,), jnp.int32),
            pltpu.VMEM((2 * C, 16), jnp.float32),
            pltpu.VMEM((2 * C, 16), jnp.float32),
            pltpu.VMEM((32,), jnp.float32),
            pltpu.VMEM((16,), jnp.float32),
            pltpu.VMEM((32, 16), jnp.float32),
            pltpu.VMEM((16,), jnp.float32),
            pltpu.SemaphoreType.DMA,
            pltpu.SemaphoreType.DMA,
            pltpu.SemaphoreType.DMA,
            pltpu.SemaphoreType.DMA,
            pltpu.SemaphoreType.DMA,
        ],
    )


def kernel(pred, gt, heatmap):
    out, _ = _sc_kernel()(
        heatmap.reshape(-1), pred.reshape(-1), gt.reshape(-1))
    return out[:B]


# speedup vs baseline: 1.0519x; 1.0519x over previous
import functools

import jax
import jax.numpy as jnp
from jax import lax
from jax.experimental import pallas as pl
from jax.experimental.pallas import tpu as pltpu
from jax.experimental.pallas import tpu_sc as plsc

B, K, H, W = 8, 100, 128, 128
HW = H * W
C = 7
SPT = 25
UNROLL = 8
NITER = HW // (16 * UNROLL)


def _merge(mv, iv, mv2, iv2):
    take = (mv2 > mv) | ((mv2 == mv) & (iv2 < iv))
    return jnp.where(take, mv2, mv), jnp.where(take, iv2, iv)


NBUF = 4


def _sc_body(hm_hbm, pred_hbm, gt_hbm, out_hbm, parts_hbm,
             hbuf, vms_v, idxs_v, gt_v, val_v,
             part_v, fold_v, red_v, out_v, sem0, sem1, sem2, sem3, semg):
    cid = lax.axis_index("c")
    sid = lax.axis_index("s")
    w = sid * 2 + cid
    base = w * SPT
    b_img = w // 4
    it = lax.broadcasted_iota(jnp.int32, (16,), 0)
    sems = (sem0, sem1, sem2, sem3)

    def row_copy(r):
        return pltpu.async_copy(
            hm_hbm.at[pl.ds((base + r) * HW, HW)], hbuf.at[r % NBUF],
            sems[r % NBUF])

    pending = {r: row_copy(r) for r in range(NBUF)}
    neg_inf = jnp.full((16,), -jnp.inf, jnp.float32)
    zero_i = jnp.zeros((16,), jnp.int32)
    vm_g = [jnp.zeros((16,), jnp.float32)] * 2
    pb_g = [zero_i] * 2

    idx0 = base + it
    idx1 = base + jnp.minimum(16 + it, SPT - 1)
    copies = []
    for g, idxg in enumerate((idx0, idx1)):
        for c in range(C):
            copies.append(
                pltpu.async_copy(gt_hbm.at[idxg * 8 + c], gt_v.at[g * C + c], semg))

    for r in range(SPT):
        pending.pop(r).wait()
        if r + NBUF < SPT:
            pending[r + NBUF] = row_copy(r + NBUF)

        ms = [neg_inf] * UNROLL
        ids = [zero_i] * UNROLL

        def body(i, carry):
            ms, ids = carry
            ms, ids = list(ms), list(ids)
            iv = jnp.broadcast_to(i, (16,)).astype(jnp.int32)
            off = i * (16 * UNROLL)
            for j in range(UNROLL):
                v = hbuf[r % NBUF, pl.ds(off + j * 16, 16)]
                take = v > ms[j]
                ms[j] = jnp.where(take, v, ms[j])
                ids[j] = jnp.where(take, iv, ids[j])
            return tuple(ms), tuple(ids)

        ms, ids = lax.fori_loop(0, NITER, body, (tuple(ms), tuple(ids)))
        mv, iv = ms[0], ids[0] * (16 * UNROLL) + it
        for j in range(1, UNROLL):
            mv, iv = _merge(mv, iv, ms[j],
                            ids[j] * (16 * UNROLL) + (j * 16) + it)

        vms_v[pl.ds(0, 16)] = mv
        idxs_v[pl.ds(0, 16)] = iv
        for sh in (8, 4, 2, 1):
            av = vms_v[pl.ds(0, 16)]
            ai = idxs_v[pl.ds(0, 16)]
            bv = vms_v[pl.ds(sh, 16)]
            bi = idxs_v[pl.ds(sh, 16)]
            av, ai = _merge(av, ai, bv, bi)
            vms_v[pl.ds(0, 16)] = av
            idxs_v[pl.ds(0, 16)] = ai
        mval = vms_v[pl.ds(0, 16)][0]
        midx = idxs_v[pl.ds(0, 16)][0] + b_img * (8 * HW)
        g, lane = (0, r) if r < 16 else (1, r - 16)
        vm_g[g] = jnp.where(it == lane, jnp.broadcast_to(mval, (16,)), vm_g[g])
        pb_g[g] = jnp.where(it == lane, jnp.broadcast_to(midx, (16,)), pb_g[g])
        if r in (15, SPT - 1):
            for c in range(C):
                copies.append(
                    pltpu.async_copy(pred_hbm.at[pb_g[g] + c * HW],
                                     val_v.at[g * C + c], semg))

    for cp in copies:
        cp.wait()

    total = jnp.zeros((16,), jnp.float32)
    for g in range(2):
        acc = jnp.zeros((16,), jnp.float32)
        for c in range(C):
            d = val_v[g * C + c] - gt_v[g * C + c]
            acc = acc + d * d
        live = vm_g[g] == 1.0
        if g == 1:
            live = live & (it < SPT - 16)
        total = total + jnp.where(live, acc, jnp.float32(0.0))

    part_v[pl.ds(0, 16)] = total
    for sh in (8, 4, 2, 1):
        part_v[pl.ds(0, 16)] = part_v[pl.ds(0, 16)] + part_v[pl.ds(sh, 16)]
    fold_v[...] = part_v[pl.ds(0, 16)]
    pltpu.sync_copy(fold_v, parts_hbm.at[w])
    plsc.subcore_barrier()

    @pl.when(w == 0)
    def _():
        pltpu.sync_copy(parts_hbm, red_v)
        vec = jnp.zeros((16,), jnp.float32)
        for bb in range(B):
            rs = (red_v[4 * bb] + red_v[4 * bb + 1]
                  + red_v[4 * bb + 2] + red_v[4 * bb + 3])
            vec = jnp.where(it == bb, jnp.broadcast_to(rs[0], (16,)), vec)
        out_v[...] = vec
        pltpu.sync_copy(out_v, out_hbm)


@functools.cache
def _sc_kernel():
    mesh = plsc.VectorSubcoreMesh(core_axis_name="c", subcore_axis_name="s")
    return pl.kernel(
        _sc_body,
        out_type=(jax.ShapeDtypeStruct((16,), jnp.float32),
                  jax.ShapeDtypeStruct((32, 16), jnp.float32)),
        mesh=mesh,
        scratch_types=[
            pltpu.VMEM((NBUF, HW), jnp.float32),
            pltpu.VMEM((32,), jnp.float32),
            pltpu.VMEM((32,), jnp.int32),
            pltpu.VMEM((2 * C, 16), jnp.float32),
            pltpu.VMEM((2 * C, 16), jnp.float32),
            pltpu.VMEM((32,), jnp.float32),
            pltpu.VMEM((16,), jnp.float32),
            pltpu.VMEM((32, 16), jnp.float32),
            pltpu.VMEM((16,), jnp.float32),
            pltpu.SemaphoreType.DMA,
            pltpu.SemaphoreType.DMA,
            pltpu.SemaphoreType.DMA,
            pltpu.SemaphoreType.DMA,
            pltpu.SemaphoreType.DMA,
        ],
    )


def kernel(pred, gt, heatmap):
    out, _ = _sc_kernel()(
        heatmap.reshape(-1), pred.reshape(-1), gt.reshape(-1))
    return out[:B]
